# baseline (speedup 1.0000x reference)
import jax
import jax.numpy as jnp
from jax.experimental import pallas as pl
from jax.experimental.pallas import tpu as pltpu

_BM = 400
_NBUF = 3


def _copy(adj_hbm, buf, sems, chunk, nbuf):
    slot = jax.lax.rem(chunk, nbuf)
    return pltpu.make_async_copy(
        adj_hbm.at[pl.ds(chunk * _BM, _BM), :],
        buf.at[slot],
        sems.at[slot],
    )


def _graph_conv_kernel(adj_hbm, x_ref, w_ref, o_ref, buf, sems):
    i = pl.program_id(0)
    nsteps = pl.num_programs(0)

    @pl.when(i == 0)
    def _():
        for c in range(_NBUF - 1):
            _copy(adj_hbm, buf, sems, c, _NBUF).start()

    @pl.when(i + _NBUF - 1 < nsteps)
    def _():
        _copy(adj_hbm, buf, sems, i + _NBUF - 1, _NBUF).start()

    _copy(adj_hbm, buf, sems, i, _NBUF).wait()
    slot = jax.lax.rem(i, _NBUF)
    h = jnp.dot(buf[slot].astype(jnp.bfloat16), x_ref[...],
                preferred_element_type=jnp.float32)
    o_ref[...] = jax.lax.dot_general(
        h, w_ref[...], (((1,), (1,)), ((), ())),
        preferred_element_type=jnp.float32)


def kernel(x, adj, W):
    n, d_in = x.shape
    d_out = W.shape[0]
    xb = x.astype(jnp.bfloat16)
    return pl.pallas_call(
        _graph_conv_kernel,
        grid=(n // _BM,),
        in_specs=[
            pl.BlockSpec(memory_space=pl.ANY),
            pl.BlockSpec((n, d_in), lambda i: (0, 0)),
            pl.BlockSpec((d_out, d_in), lambda i: (0, 0)),
        ],
        out_specs=pl.BlockSpec((_BM, d_out), lambda i: (i, 0)),
        out_shape=jax.ShapeDtypeStruct((n, d_out), jnp.float32),
        scratch_shapes=[
            pltpu.VMEM((_NBUF, _BM, n), jnp.float32),
            pltpu.SemaphoreType.DMA((_NBUF,)),
        ],
        compiler_params=pltpu.CompilerParams(
            vmem_limit_bytes=120 * 1024 * 1024),
    )(adj, xb, W)


# speedup vs baseline: 1.0384x; 1.0384x over previous
import jax
import jax.numpy as jnp
from jax.experimental import pallas as pl
from jax.experimental.pallas import tpu as pltpu

_BM = 400


def _graph_conv_kernel(adj_ref, x_ref, w_ref, o_ref):
    h = jnp.dot(adj_ref[...].astype(jnp.bfloat16), x_ref[...],
                preferred_element_type=jnp.float32)
    o_ref[...] = jax.lax.dot_general(
        h, w_ref[...], (((1,), (1,)), ((), ())),
        preferred_element_type=jnp.float32)


def kernel(x, adj, W):
    n, d_in = x.shape
    d_out = W.shape[0]
    xb = x.astype(jnp.bfloat16)
    return pl.pallas_call(
        _graph_conv_kernel,
        grid=(n // _BM,),
        in_specs=[
            pl.BlockSpec((_BM, n), lambda i: (i, 0)),
            pl.BlockSpec((n, d_in), lambda i: (0, 0)),
            pl.BlockSpec((d_out, d_in), lambda i: (0, 0)),
        ],
        out_specs=pl.BlockSpec((_BM, d_out), lambda i: (i, 0)),
        out_shape=jax.ShapeDtypeStruct((n, d_out), jnp.float32),
        compiler_params=pltpu.CompilerParams(
            vmem_limit_bytes=110 * 1024 * 1024),
    )(adj, xb, W)


# speedup vs baseline: 1.0397x; 1.0013x over previous
import jax
import jax.numpy as jnp
from jax.experimental import pallas as pl
from jax.experimental.pallas import tpu as pltpu

_BM = 400


def _graph_conv_kernel(adj_ref, x_ref, w_ref, o_ref):
    h = jnp.dot(adj_ref[...].astype(jnp.bfloat16), x_ref[...],
                preferred_element_type=jnp.float32)
    o_ref[...] = jax.lax.dot_general(
        h, w_ref[...], (((1,), (1,)), ((), ())),
        preferred_element_type=jnp.float32)


def kernel(x, adj, W):
    n, d_in = x.shape
    d_out = W.shape[0]
    xb = x.astype(jnp.bfloat16)
    return pl.pallas_call(
        _graph_conv_kernel,
        grid=(n // _BM,),
        in_specs=[
            pl.BlockSpec((_BM, n), lambda i: (i, 0)),
            pl.BlockSpec((n, d_in), lambda i: (0, 0)),
            pl.BlockSpec((d_out, d_in), lambda i: (0, 0)),
        ],
        out_specs=pl.BlockSpec((_BM, d_out), lambda i: (i, 0)),
        out_shape=jax.ShapeDtypeStruct((n, d_out), jnp.float32),
        compiler_params=pltpu.CompilerParams(
            vmem_limit_bytes=110 * 1024 * 1024,
            allow_input_fusion=[False, True, False]),
    )(adj, xb, W)


# speedup vs baseline: 1.0555x; 1.0152x over previous
import jax
import jax.numpy as jnp
from jax.experimental import pallas as pl
from jax.experimental.pallas import tpu as pltpu

_BM = 400


def _graph_conv_kernel(adj_ref, x_ref, w_ref, o_ref):
    h = jnp.dot(adj_ref[...].astype(jnp.bfloat16),
                x_ref[...].astype(jnp.bfloat16),
                preferred_element_type=jnp.float32)
    o_ref[...] = jax.lax.dot_general(
        h, w_ref[...], (((1,), (1,)), ((), ())),
        preferred_element_type=jnp.float32)


def kernel(x, adj, W):
    n, d_in = x.shape
    d_out = W.shape[0]
    return pl.pallas_call(
        _graph_conv_kernel,
        grid=(n // _BM,),
        in_specs=[
            pl.BlockSpec((_BM, n), lambda i: (i, 0)),
            pl.BlockSpec((n, d_in), lambda i: (0, 0)),
            pl.BlockSpec((d_out, d_in), lambda i: (0, 0)),
        ],
        out_specs=pl.BlockSpec((_BM, d_out), lambda i: (i, 0)),
        out_shape=jax.ShapeDtypeStruct((n, d_out), jnp.float32),
        compiler_params=pltpu.CompilerParams(
            vmem_limit_bytes=110 * 1024 * 1024),
    )(adj, x, W)


# speedup vs baseline: 1.0604x; 1.0046x over previous
import jax
import jax.numpy as jnp
from jax.experimental import pallas as pl
from jax.experimental.pallas import tpu as pltpu

_BM = 400


def _graph_conv_kernel(adj_ref, x_ref, w_ref, o_ref, z_ref):
    @pl.when(pl.program_id(0) == 0)
    def _():
        z_ref[...] = jax.lax.dot_general(
            x_ref[...], w_ref[...], (((1,), (1,)), ((), ())),
            preferred_element_type=jnp.float32).astype(jnp.bfloat16)

    o_ref[...] = jnp.dot(adj_ref[...].astype(jnp.bfloat16), z_ref[...],
                         preferred_element_type=jnp.float32)


def kernel(x, adj, W):
    n, d_in = x.shape
    d_out = W.shape[0]
    return pl.pallas_call(
        _graph_conv_kernel,
        grid=(n // _BM,),
        in_specs=[
            pl.BlockSpec((_BM, n), lambda i: (i, 0)),
            pl.BlockSpec((n, d_in), lambda i: (0, 0)),
            pl.BlockSpec((d_out, d_in), lambda i: (0, 0)),
        ],
        out_specs=pl.BlockSpec((_BM, d_out), lambda i: (i, 0)),
        out_shape=jax.ShapeDtypeStruct((n, d_out), jnp.float32),
        scratch_shapes=[pltpu.VMEM((n, d_out), jnp.bfloat16)],
        compiler_params=pltpu.CompilerParams(
            vmem_limit_bytes=110 * 1024 * 1024),
    )(adj, x, W)


# speedup vs baseline: 1.0617x; 1.0012x over previous
import jax
import jax.numpy as jnp
from jax.experimental import pallas as pl
from jax.experimental.pallas import tpu as pltpu

_BM = 400


def _graph_conv_kernel(adj_ref, x_ref, w_ref, o_ref):
    h = jnp.dot(adj_ref[...].astype(jnp.bfloat16),
                x_ref[...].astype(jnp.bfloat16),
                preferred_element_type=jnp.float32)
    o_ref[...] = jax.lax.dot_general(
        h, w_ref[...], (((1,), (1,)), ((), ())),
        preferred_element_type=jnp.float32)


def kernel(x, adj, W):
    n, d_in = x.shape
    d_out = W.shape[0]
    return pl.pallas_call(
        _graph_conv_kernel,
        grid=(n // _BM,),
        in_specs=[
            pl.BlockSpec((_BM, n), lambda i: (i, 0)),
            pl.BlockSpec((n, d_in), lambda i: (0, 0)),
            pl.BlockSpec((d_out, d_in), lambda i: (0, 0)),
        ],
        out_specs=pl.BlockSpec((_BM, d_out), lambda i: (i, 0)),
        out_shape=jax.ShapeDtypeStruct((n, d_out), jnp.float32),
        compiler_params=pltpu.CompilerParams(
            vmem_limit_bytes=110 * 1024 * 1024),
    )(adj, x, W)
